# baseline (speedup 1.0000x reference)
import functools

import jax
import jax.numpy as jnp
from jax import lax
from jax.experimental import pallas as pl
from jax.experimental.pallas import tpu as pltpu
from jax.experimental.pallas import tpu_sc as plsc

_EPS = 1e-12
_L = 16


def _rsqrt_newton(x):
    i = plsc.bitcast(x, jnp.int32)
    i = jnp.int32(0x5F3759DF) - lax.shift_right_logical(i, 1)
    y = plsc.bitcast(i, jnp.float32)
    for _ in range(3):
        y = y * (1.5 - 0.5 * x * y * y)
    return y


def _make_ln_embed(N, V, H, NC, NS):
    NW = NC * NS
    b_per_w = N // NW
    CH = 64
    n_ch = b_per_w // CH
    n_grp = CH // _L
    UN1 = 8
    UN2 = 4

    mesh = plsc.VectorSubcoreMesh(core_axis_name="c", subcore_axis_name="s")

    @functools.partial(
        pl.kernel,
        mesh=mesh,
        compiler_params=pltpu.CompilerParams(use_tc_tiling_on_sc=False,
                                             needs_layout_passes=False),
        out_type=jax.ShapeDtypeStruct((N, H), jnp.float32),
        scratch_types=[
            pltpu.VMEM((CH,), jnp.int32),
            pltpu.VMEM((CH, H), jnp.float32),
            pltpu.VMEM((H,), jnp.float32),
            pltpu.VMEM((H,), jnp.float32),
            pltpu.SemaphoreType.DMA,
        ],
    )
    def ln_embed(ids_hbm, table_hbm, gamma_hbm, beta_hbm, out_hbm,
                 idx_v, rows_v, g_v, b_v, sem):
        wid = lax.axis_index("s") * NC + lax.axis_index("c")
        base = wid * b_per_w
        pltpu.sync_copy(gamma_hbm, g_v)
        pltpu.sync_copy(beta_hbm, b_v)
        lane = lax.broadcasted_iota(jnp.int32, (_L,), 0)

        def chunk_body(c, _):
            cb = base + c * CH
            pltpu.sync_copy(ids_hbm.at[pl.ds(cb, CH)], idx_v)
            pltpu.async_copy(table_hbm.at[idx_v], rows_v, sem).wait()

            stats = []
            for g in range(n_grp):
                row_ids = lane + g * _L

                def acc_body(j, carry, row_ids=row_ids):
                    s0, s1, q0, q1 = carry
                    for u in range(UN1):
                        col = jnp.full((_L,), j * UN1 + u, jnp.int32)
                        x = plsc.load_gather(rows_v, [row_ids, col])
                        if u % 2 == 0:
                            s0 = s0 + x
                            q0 = q0 + x * x
                        else:
                            s1 = s1 + x
                            q1 = q1 + x * x
                    return s0, s1, q0, q1

                zero = jnp.zeros((_L,), jnp.float32)
                s0, s1, q0, q1 = lax.fori_loop(
                    0, H // UN1, acc_body, (zero, zero, zero, zero))
                mean = (s0 + s1) * (1.0 / H)
                var = (q0 + q1) * (1.0 / H) - mean * mean
                rstd = _rsqrt_newton(var + _EPS)
                nm = -(mean * rstd)
                stats.append((row_ids, rstd, nm))

            def norm_body(j, _):
                for u in range(UN2):
                    cc = j * UN2 + u
                    col = jnp.full((_L,), cc, jnp.int32)
                    gs = plsc.load_gather(g_v, [col])
                    bs = plsc.load_gather(b_v, [col])
                    for row_ids, rstd, nm in stats:
                        x = plsc.load_gather(rows_v, [row_ids, col])
                        t = x * rstd + nm
                        plsc.store_scatter(rows_v, [row_ids, col],
                                           t * gs + bs)
                return 0

            lax.fori_loop(0, H // UN2, norm_body, 0)
            pltpu.sync_copy(rows_v, out_hbm.at[pl.ds(cb, CH)])
            return 0

        lax.fori_loop(0, n_ch, chunk_body, 0)

    return ln_embed


def kernel(input_ids, table, gamma, beta):
    B, S = input_ids.shape
    V, H = table.shape
    N = B * S
    info = plsc.get_sparse_core_info()
    ids = input_ids.reshape(N).astype(jnp.int32)
    ln_embed = _make_ln_embed(N, V, H, info.num_cores, info.num_subcores)
    out = ln_embed(ids, table, gamma, beta)
    return out.reshape(B, S, H)


# speedup vs baseline: 2.4883x; 2.4883x over previous
import functools

import jax
import jax.numpy as jnp
from jax import lax
from jax.experimental import pallas as pl
from jax.experimental.pallas import tpu as pltpu
from jax.experimental.pallas import tpu_sc as plsc

_EPS = 1e-12
_L = 16


def _dyn_gather(x, idx):
    dnums = lax.GatherDimensionNumbers(
        offset_dims=(), collapsed_slice_dims=(0,), start_index_map=(0,))
    return lax.gather(x, idx[:, None], dnums, (1,),
                      mode=lax.GatherScatterMode.PROMISE_IN_BOUNDS)


def _rsqrt_newton(x):
    i = plsc.bitcast(x, jnp.int32)
    i = jnp.int32(0x5F3759DF) - lax.shift_right_logical(i, 1)
    y = plsc.bitcast(i, jnp.float32)
    for _ in range(3):
        y = y * (1.5 - 0.5 * x * y * y)
    return y


def _make_ln_embed(N, V, H, NC, NS):
    NW = NC * NS
    b_per_w = N // NW
    CH = 64
    n_ch = b_per_w // CH
    n_grp = CH // _L
    J = H // _L

    mesh = plsc.VectorSubcoreMesh(core_axis_name="c", subcore_axis_name="s")

    @functools.partial(
        pl.kernel,
        mesh=mesh,
        compiler_params=pltpu.CompilerParams(use_tc_tiling_on_sc=False,
                                             needs_layout_passes=False),
        out_type=jax.ShapeDtypeStruct((N, H), jnp.float32),
        scratch_types=[
            pltpu.VMEM((CH,), jnp.int32),
            pltpu.VMEM((CH, H), jnp.float32),
            pltpu.VMEM((CH,), jnp.float32),
            pltpu.VMEM((CH,), jnp.float32),
            pltpu.VMEM((H,), jnp.float32),
            pltpu.VMEM((H,), jnp.float32),
            pltpu.SemaphoreType.DMA,
        ],
    )
    def ln_embed(ids_hbm, table_hbm, gamma_hbm, beta_hbm, out_hbm,
                 idx_v, rows_v, ssum_v, sq_v, g_v, b_v, sem):
        wid = lax.axis_index("s") * NC + lax.axis_index("c")
        base = wid * b_per_w
        pltpu.sync_copy(gamma_hbm, g_v)
        pltpu.sync_copy(beta_hbm, b_v)
        lane = lax.broadcasted_iota(jnp.int32, (_L,), 0)
        bperm = [jnp.bitwise_xor(lane, k) for k in (1, 2, 4, 8)]

        def chunk_body(c, _):
            cb = base + c * CH
            pltpu.sync_copy(ids_hbm.at[pl.ds(cb, CH)], idx_v)
            pltpu.async_copy(table_hbm.at[idx_v], rows_v, sem).wait()

            def stat_body(r, _):
                s = [jnp.zeros((_L,), jnp.float32) for _ in range(3)]
                q = [jnp.zeros((_L,), jnp.float32) for _ in range(3)]
                for j in range(J):
                    x = rows_v[r, pl.ds(j * _L, _L)]
                    s[j % 3] = s[j % 3] + x
                    q[j % 3] = q[j % 3] + x * x
                st = s[0] + s[1] + s[2]
                qt = q[0] + q[1] + q[2]
                for p in bperm:
                    st = st + _dyn_gather(st, p)
                for p in bperm:
                    qt = qt + _dyn_gather(qt, p)
                ridx = jnp.full((_L,), r, jnp.int32)
                m0 = lane == 0
                plsc.store_scatter(ssum_v, [ridx], st, mask=m0)
                plsc.store_scatter(sq_v, [ridx], qt, mask=m0)
                return 0

            lax.fori_loop(0, CH, stat_body, 0)

            for g in range(n_grp):
                sv = ssum_v[pl.ds(g * _L, _L)]
                qv = sq_v[pl.ds(g * _L, _L)]
                mean = sv * (1.0 / H)
                var = qv * (1.0 / H) - mean * mean
                rstd = _rsqrt_newton(var + _EPS)
                nm = -(mean * rstd)
                rs = [_dyn_gather(rstd, jnp.full((_L,), rr, jnp.int32))
                      for rr in range(_L)]
                nms = [_dyn_gather(nm, jnp.full((_L,), rr, jnp.int32))
                       for rr in range(_L)]

                def norm_body(j, _, g=g, rs=rs, nms=nms):
                    sl = pl.ds(pl.multiple_of(j * _L, _L), _L)
                    gv = g_v[sl]
                    bv = b_v[sl]
                    for rr in range(_L):
                        row = g * _L + rr
                        x = rows_v[row, sl]
                        rows_v[row, sl] = (x * rs[rr] + nms[rr]) * gv + bv
                    return 0

                lax.fori_loop(0, J, norm_body, 0)

            pltpu.sync_copy(rows_v, out_hbm.at[pl.ds(cb, CH)])
            return 0

        lax.fori_loop(0, n_ch, chunk_body, 0)

    return ln_embed


def kernel(input_ids, table, gamma, beta):
    B, S = input_ids.shape
    V, H = table.shape
    N = B * S
    info = plsc.get_sparse_core_info()
    ids = input_ids.reshape(N).astype(jnp.int32)
    ln_embed = _make_ln_embed(N, V, H, info.num_cores, info.num_subcores)
    out = ln_embed(ids, table, gamma, beta)
    return out.reshape(B, S, H)


# speedup vs baseline: 9.5555x; 3.8401x over previous
import functools

import jax
import jax.numpy as jnp
from jax import lax
from jax.experimental import pallas as pl
from jax.experimental.pallas import tpu as pltpu
from jax.experimental.pallas import tpu_sc as plsc

_EPS = 1e-12
_L = 16


def _dyn_gather(x, idx):
    dnums = lax.GatherDimensionNumbers(
        offset_dims=(), collapsed_slice_dims=(0,), start_index_map=(0,))
    return lax.gather(x, idx[:, None], dnums, (1,),
                      mode=lax.GatherScatterMode.PROMISE_IN_BOUNDS)


def _rsqrt_newton(x):
    i = plsc.bitcast(x, jnp.int32)
    i = jnp.int32(0x5F3759DF) - lax.shift_right_logical(i, 1)
    y = plsc.bitcast(i, jnp.float32)
    for _ in range(3):
        y = y * (1.5 - 0.5 * x * y * y)
    return y


def _make_ln_embed(N, V, H, NC, NS):
    NW = NC * NS
    b_per_w = N // NW
    CH = 64
    n_ch = b_per_w // CH
    n_grp = CH // _L
    J = H // _L

    mesh = plsc.VectorSubcoreMesh(core_axis_name="c", subcore_axis_name="s")

    @functools.partial(
        pl.kernel,
        mesh=mesh,
        compiler_params=pltpu.CompilerParams(use_tc_tiling_on_sc=True,
                                             needs_layout_passes=False),
        out_type=jax.ShapeDtypeStruct((N, H), jnp.float32),
        scratch_types=[
            pltpu.VMEM((CH,), jnp.int32),
            pltpu.VMEM((CH, H), jnp.float32),
            pltpu.VMEM((CH,), jnp.float32),
            pltpu.VMEM((CH,), jnp.float32),
            pltpu.VMEM((H,), jnp.float32),
            pltpu.VMEM((H,), jnp.float32),
            pltpu.SemaphoreType.DMA,
        ],
    )
    def ln_embed(ids_hbm, table_hbm, gamma_hbm, beta_hbm, out_hbm,
                 idx_v, rows_v, ssum_v, sq_v, g_v, b_v, sem):
        wid = lax.axis_index("s") * NC + lax.axis_index("c")
        base = wid * b_per_w
        pltpu.sync_copy(gamma_hbm, g_v)
        pltpu.sync_copy(beta_hbm, b_v)
        lane = lax.broadcasted_iota(jnp.int32, (_L,), 0)
        bperm = [jnp.bitwise_xor(lane, k) for k in (1, 2, 4, 8)]

        def chunk_body(c, _):
            cb = base + c * CH
            pltpu.sync_copy(ids_hbm.at[pl.ds(cb, CH)], idx_v)
            pltpu.async_copy(table_hbm.at[idx_v], rows_v, sem).wait()

            def stat_body(r, _):
                s = [jnp.zeros((_L,), jnp.float32) for _ in range(3)]
                q = [jnp.zeros((_L,), jnp.float32) for _ in range(3)]
                for j in range(J):
                    x = rows_v[r, pl.ds(j * _L, _L)]
                    s[j % 3] = s[j % 3] + x
                    q[j % 3] = q[j % 3] + x * x
                st = s[0] + s[1] + s[2]
                qt = q[0] + q[1] + q[2]
                for p in bperm:
                    st = st + _dyn_gather(st, p)
                for p in bperm:
                    qt = qt + _dyn_gather(qt, p)
                ridx = jnp.full((_L,), r, jnp.int32)
                m0 = lane == 0
                plsc.store_scatter(ssum_v, [ridx], st, mask=m0)
                plsc.store_scatter(sq_v, [ridx], qt, mask=m0)
                return 0

            lax.fori_loop(0, CH, stat_body, 0)

            for g in range(n_grp):
                sv = ssum_v[pl.ds(g * _L, _L)]
                qv = sq_v[pl.ds(g * _L, _L)]
                mean = sv * (1.0 / H)
                var = qv * (1.0 / H) - mean * mean
                rstd = _rsqrt_newton(var + _EPS)
                nm = -(mean * rstd)
                rs = [_dyn_gather(rstd, jnp.full((_L,), rr, jnp.int32))
                      for rr in range(_L)]
                nms = [_dyn_gather(nm, jnp.full((_L,), rr, jnp.int32))
                       for rr in range(_L)]

                def norm_body(j, _, g=g, rs=rs, nms=nms):
                    sl = pl.ds(pl.multiple_of(j * _L, _L), _L)
                    gv = g_v[sl]
                    bv = b_v[sl]
                    for rr in range(_L):
                        row = g * _L + rr
                        x = rows_v[row, sl]
                        rows_v[row, sl] = (x * rs[rr] + nms[rr]) * gv + bv
                    return 0

                lax.fori_loop(0, J, norm_body, 0)

            pltpu.sync_copy(rows_v, out_hbm.at[pl.ds(cb, CH)])
            return 0

        lax.fori_loop(0, n_ch, chunk_body, 0)

    return ln_embed


def kernel(input_ids, table, gamma, beta):
    B, S = input_ids.shape
    V, H = table.shape
    N = B * S
    info = plsc.get_sparse_core_info()
    ids = input_ids.reshape(N).astype(jnp.int32)
    ln_embed = _make_ln_embed(N, V, H, info.num_cores, info.num_subcores)
    out = ln_embed(ids, table, gamma, beta)
    return out.reshape(B, S, H)


# speedup vs baseline: 12.1757x; 1.2742x over previous
import functools

import jax
import jax.numpy as jnp
from jax import lax
from jax.experimental import pallas as pl
from jax.experimental.pallas import tpu as pltpu
from jax.experimental.pallas import tpu_sc as plsc

_EPS = 1e-12
_L = 16


def _dyn_gather(x, idx):
    dnums = lax.GatherDimensionNumbers(
        offset_dims=(), collapsed_slice_dims=(0,), start_index_map=(0,))
    return lax.gather(x, idx[:, None], dnums, (1,),
                      mode=lax.GatherScatterMode.PROMISE_IN_BOUNDS)


def _rsqrt_newton(x):
    i = plsc.bitcast(x, jnp.int32)
    i = jnp.int32(0x5F3759DF) - lax.shift_right_logical(i, 1)
    y = plsc.bitcast(i, jnp.float32)
    for _ in range(3):
        y = y * (1.5 - 0.5 * x * y * y)
    return y


def _make_ln_embed(N, V, H, NC, NS):
    NW = NC * NS
    b_per_w = N // NW
    CH = 32
    n_ch = b_per_w // CH
    n_grp = CH // _L
    J = H // _L
    NB = 4

    mesh = plsc.VectorSubcoreMesh(core_axis_name="c", subcore_axis_name="s")

    @functools.partial(
        pl.kernel,
        mesh=mesh,
        compiler_params=pltpu.CompilerParams(use_tc_tiling_on_sc=True,
                                             needs_layout_passes=False),
        out_type=jax.ShapeDtypeStruct((N, H), jnp.float32),
        scratch_types=[
            pltpu.VMEM((n_ch, CH), jnp.int32),
            pltpu.VMEM((NB, CH, H), jnp.float32),
            pltpu.VMEM((CH,), jnp.float32),
            pltpu.VMEM((CH,), jnp.float32),
            pltpu.VMEM((H,), jnp.float32),
            pltpu.VMEM((H,), jnp.float32),
        ] + [pltpu.SemaphoreType.DMA] * (2 * NB),
    )
    def ln_embed(ids_hbm, table_hbm, gamma_hbm, beta_hbm, out_hbm,
                 idx_v, rows_v, ssum_v, sq_v, g_v, b_v, *sems):
        sem_g = sems[:NB]
        sem_o = sems[NB:]
        wid = lax.axis_index("s") * NC + lax.axis_index("c")
        base = wid * b_per_w
        pltpu.sync_copy(ids_hbm.at[wid], idx_v)
        pltpu.sync_copy(gamma_hbm, g_v)
        pltpu.sync_copy(beta_hbm, b_v)
        lane = lax.broadcasted_iota(jnp.int32, (_L,), 0)
        bperm = [jnp.bitwise_xor(lane, k) for k in (1, 2, 4, 8)]

        def gather_copy(c, buf):
            return pltpu.make_async_copy(
                table_hbm.at[idx_v.at[c]],
                rows_v.at[buf], sem_g[buf])

        def out_copy(c, buf):
            off = pl.multiple_of(base + c * CH, CH)
            return pltpu.make_async_copy(
                rows_v.at[buf], out_hbm.at[pl.ds(off, CH)], sem_o[buf])

        for k in range(2):
            gather_copy(jnp.int32(k), k).start()

        def outer(c4, _):
            for b in range(NB):
                c = c4 * NB + b
                gather_copy(c, b).wait()

                cn = c + 2
                bn = (b + 2) % NB

                @pl.when(cn < n_ch)
                def _():
                    @pl.when(cn >= NB)
                    def _():
                        out_copy(cn - NB, bn).wait()
                    gather_copy(cn, bn).start()

                rows = rows_v.at[b]

                def stat_body(r, _):
                    s = [jnp.zeros((_L,), jnp.float32) for _ in range(3)]
                    q = [jnp.zeros((_L,), jnp.float32) for _ in range(3)]
                    for j in range(J):
                        x = rows[r, pl.ds(j * _L, _L)]
                        s[j % 3] = s[j % 3] + x
                        q[j % 3] = q[j % 3] + x * x
                    st = s[0] + s[1] + s[2]
                    qt = q[0] + q[1] + q[2]
                    for p in bperm:
                        st = st + _dyn_gather(st, p)
                    for p in bperm:
                        qt = qt + _dyn_gather(qt, p)
                    ridx = jnp.full((_L,), r, jnp.int32)
                    m0 = lane == 0
                    plsc.store_scatter(ssum_v, [ridx], st, mask=m0)
                    plsc.store_scatter(sq_v, [ridx], qt, mask=m0)
                    return 0

                lax.fori_loop(0, CH, stat_body, 0)

                for g in range(n_grp):
                    sv = ssum_v[pl.ds(g * _L, _L)]
                    qv = sq_v[pl.ds(g * _L, _L)]
                    mean = sv * (1.0 / H)
                    var = qv * (1.0 / H) - mean * mean
                    rstd = _rsqrt_newton(var + _EPS)
                    nm = -(mean * rstd)
                    rs = [_dyn_gather(rstd, jnp.full((_L,), rr, jnp.int32))
                          for rr in range(_L)]
                    nms = [_dyn_gather(nm, jnp.full((_L,), rr, jnp.int32))
                           for rr in range(_L)]

                    def norm_body(j, _, g=g, rs=rs, nms=nms, rows=rows):
                        sl = pl.ds(pl.multiple_of(j * _L, _L), _L)
                        gv = g_v[sl]
                        bv = b_v[sl]
                        for rr in range(_L):
                            row = g * _L + rr
                            x = rows[row, sl]
                            rows[row, sl] = (x * rs[rr] + nms[rr]) * gv + bv
                        return 0

                    lax.fori_loop(0, J, norm_body, 0)

                out_copy(c, b).start()
            return 0

        lax.fori_loop(0, n_ch // NB, outer, 0)

        for b in range(NB):
            c = n_ch - NB + b
            out_copy(jnp.int32(c), b).wait()

    return ln_embed


def kernel(input_ids, table, gamma, beta):
    B, S = input_ids.shape
    V, H = table.shape
    N = B * S
    info = plsc.get_sparse_core_info()
    NW = info.num_cores * info.num_subcores
    ids = input_ids.reshape(NW, -1, 32).astype(jnp.int32)
    ln_embed = _make_ln_embed(N, V, H, info.num_cores, info.num_subcores)
    out = ln_embed(ids, table, gamma, beta)
    return out.reshape(B, S, H)
